# Initial kernel scaffold; baseline (speedup 1.0000x reference)
import jax
import jax.numpy as jnp
from jax.experimental import pallas as pl


def kernel(x, edge_index, edge_attr, We1, be1, W1l, b1l, W1r, We2, be2, W2l, b2l, W2r, Wp1, bp1, Wp2, bp2):
    raise NotImplementedError("write your pallas kernel here")


# speedup vs baseline: 4.0038x; 4.0038x over previous
import functools

import jax
import jax.numpy as jnp
from jax import lax
from jax.experimental import pallas as pl
from jax.experimental.pallas import tpu as pltpu
from jax.experimental.pallas import tpu_sc as plsc

N = 10000
NP = 10240
E = 320000
D = 128
DE = 16
H = 128
C = 2

NC = 2
NS = 16
NW = NC * NS
EPT = E // NW
SUB = 40
GPC = 5
CHUNK = SUB * GPC
NCHUNK = EPT // CHUNK
NCH_TOT = E // CHUNK
RPT = NP // NS

_mesh = functools.partial(
    plsc.VectorSubcoreMesh, core_axis_name="c", subcore_axis_name="s")


def _sc_spmm():
  def body(table, src3d, dst3d, z128, sx_out, accx, sidx, didx, rows, sem):
    c = lax.axis_index("c")
    s = lax.axis_index("s")
    nbase = s * RPT
    pltpu.sync_copy(z128.at[pl.ds(nbase, RPT)], accx.at[pl.ds(nbase, RPT)])
    plsc.subcore_barrier()

    cbase = (c * NS + s) * NCHUNK

    @pl.loop(0, NCHUNK)
    def _(ch):
      cid = cbase + ch
      pltpu.sync_copy(src3d.at[cid], sidx)
      pltpu.sync_copy(dst3d.at[cid], didx)
      descs = [
          pltpu.async_copy(table.at[sidx.at[j]],
                           rows.at[pl.ds(j * SUB, SUB)], sem)
          for j in range(GPC)
      ]
      for d in descs:
        d.wait()
      for j in range(GPC):
        pltpu.sync_copy(rows.at[pl.ds(j * SUB, SUB)],
                        accx.at[didx.at[j]], add=True)

    plsc.subcore_barrier()
    pltpu.sync_copy(accx.at[pl.ds(nbase, RPT)],
                    sx_out.at[c, pl.ds(nbase, RPT)])

  return pl.kernel(
      body,
      out_type=jax.ShapeDtypeStruct((NC, NP, D), jnp.float32),
      mesh=_mesh(),
      scratch_types=[
          pltpu.VMEM_SHARED((NP, D), jnp.float32),
          pltpu.VMEM((GPC, SUB), jnp.int32),
          pltpu.VMEM((GPC, SUB), jnp.int32),
          pltpu.VMEM((CHUNK, D), jnp.float32),
          pltpu.SemaphoreType.DMA,
      ],
      name="sc_spmm")


EA2 = 2 * DE


def _sc_ea_cnt():
  def body(ea2, dst3d, z128, acc_out, acc, didx, eav, eaw, sem):
    c = lax.axis_index("c")
    s = lax.axis_index("s")
    nbase = s * RPT
    pltpu.sync_copy(z128.at[pl.ds(nbase, RPT)], acc.at[pl.ds(nbase, RPT)])
    pltpu.sync_copy(z128.at[pl.ds(0, SUB)], eaw)
    plsc.subcore_barrier()

    cbase = (c * NS + s) * NCHUNK

    @pl.loop(0, NCHUNK)
    def _(ch):
      cid = cbase + ch
      pltpu.sync_copy(dst3d.at[cid], didx)
      pltpu.sync_copy(ea2.at[pl.ds(cid * CHUNK, CHUNK)], eav)

      for j in range(GPC):
        @pl.loop(0, SUB // 4)
        def _(r4):
          for u in range(4):
            r = r4 * 4 + u
            e = j * SUB + r
            eaw[r, pl.ds(0, 16)] = eav[e, pl.ds(0, 16)]
            eaw[r, pl.ds(16, 16)] = eav[e, pl.ds(16, 16)]
        pltpu.sync_copy(eaw, acc.at[didx.at[j]], add=True)

    plsc.subcore_barrier()
    pltpu.sync_copy(acc.at[pl.ds(nbase, RPT)],
                    acc_out.at[c, pl.ds(nbase, RPT)])

  return pl.kernel(
      body,
      out_type=jax.ShapeDtypeStruct((NC, NP, D), jnp.float32),
      mesh=_mesh(),
      scratch_types=[
          pltpu.VMEM_SHARED((NP, D), jnp.float32),
          pltpu.VMEM((GPC, SUB), jnp.int32),
          pltpu.VMEM((CHUNK, EA2), jnp.float32),
          pltpu.VMEM((SUB, D), jnp.float32),
          pltpu.SemaphoreType.DMA,
      ],
      name="sc_ea_cnt")


def _sc_edge_q():
  def body(ps, pd, src3d, dst3d, q_out, sidx, didx, rows, sem):
    c = lax.axis_index("c")
    s = lax.axis_index("s")
    cbase = (c * NS + s) * NCHUNK

    @pl.loop(0, NCHUNK)
    def _(ch):
      cid = cbase + ch
      pltpu.sync_copy(src3d.at[cid], sidx)
      pltpu.sync_copy(dst3d.at[cid], didx)
      descs = [
          pltpu.async_copy(ps.at[sidx.at[j]],
                           rows.at[pl.ds(j * SUB, SUB)], sem)
          for j in range(GPC)
      ]
      for d in descs:
        d.wait()
      descs = [
          pltpu.async_copy(pd.at[didx.at[j]],
                           rows.at[pl.ds(j * SUB, SUB)], sem, add=True)
          for j in range(GPC)
      ]
      for d in descs:
        d.wait()
      pltpu.sync_copy(rows, q_out.at[pl.ds(cid * CHUNK, CHUNK)])

  return pl.kernel(
      body,
      out_type=jax.ShapeDtypeStruct((E, H), jnp.float32),
      mesh=_mesh(),
      scratch_types=[
          pltpu.VMEM((GPC, SUB), jnp.int32),
          pltpu.VMEM((GPC, SUB), jnp.int32),
          pltpu.VMEM((CHUNK, H), jnp.float32),
          pltpu.SemaphoreType.DMA,
      ],
      name="sc_edge_q")


def _tc_node_stage(sx0, sx1, eacnt0, eacnt1, x_in, We, be, Wl, bl, Wr,
                   Wp_top=None, Wp_bot=None):
  with_proj = Wp_top is not None

  def body(*refs):
    if with_proj:
      (sx0r, sx1r, e0r, e1r, xr, Wer, ber, Wlr, blr, Wrr,
       Wtr, Wbr, hr, psr, pdr) = refs
    else:
      (sx0r, sx1r, e0r, e1r, xr, Wer, ber, Wlr, blr, Wrr, hr) = refs
    eacnt = e0r[...] + e1r[...]
    Ea = eacnt[:, :DE]
    cnt = eacnt[:, DE:DE + 1]
    denom = jnp.maximum(cnt, 1.0)
    S = sx0r[...] + sx1r[...]
    msum = S + jnp.dot(Ea, Wer[...],
                       preferred_element_type=jnp.float32) + cnt * ber[...]
    agg = msum / denom
    h = jnp.maximum(
        jnp.dot(agg, Wlr[...], preferred_element_type=jnp.float32)
        + blr[...]
        + jnp.dot(xr[...], Wrr[...], preferred_element_type=jnp.float32),
        0.0)
    hr[...] = h
    if with_proj:
      psr[...] = jnp.dot(h, Wtr[...], preferred_element_type=jnp.float32)
      pdr[...] = jnp.dot(h, Wbr[...], preferred_element_type=jnp.float32)

  out_shape = jax.ShapeDtypeStruct((NP, H), jnp.float32)
  if with_proj:
    out_shape = (out_shape,
                 jax.ShapeDtypeStruct((NP, H), jnp.float32),
                 jax.ShapeDtypeStruct((NP, H), jnp.float32))
  args = [sx0, sx1, eacnt0, eacnt1, x_in, We, be, Wl, bl, Wr]
  if with_proj:
    args += [Wp_top, Wp_bot]
  return pl.pallas_call(body, out_shape=out_shape,
                        name="tc_node_proj" if with_proj else "tc_node")(*args)


_EB = 8000


def _tc_pred_tail(q, bp1, Wp2, bp2):
  def body(qr, b1r, W2r, b2r, outr):
    z = jnp.maximum(qr[...] + b1r[...], 0.0)
    outr[...] = jnp.dot(z, W2r[...],
                        preferred_element_type=jnp.float32) + b2r[...]

  return pl.pallas_call(
      body,
      grid=(E // _EB,),
      in_specs=[
          pl.BlockSpec((_EB, H), lambda i: (i, 0)),
          pl.BlockSpec((H,), lambda i: (0,)),
          pl.BlockSpec((H, C), lambda i: (0, 0)),
          pl.BlockSpec((C,), lambda i: (0,)),
      ],
      out_specs=pl.BlockSpec((_EB, C), lambda i: (i, 0)),
      out_shape=jax.ShapeDtypeStruct((E, C), jnp.float32),
      name="tc_pred_tail")(q, bp1, Wp2, bp2)


@jax.jit
def kernel(x, edge_index, edge_attr, We1, be1, W1l, b1l, W1r,
           We2, be2, W2l, b2l, W2r, Wp1, bp1, Wp2, bp2):
  src3d = edge_index[0].astype(jnp.int32).reshape(NCH_TOT, GPC, SUB)
  dst3d = edge_index[1].astype(jnp.int32).reshape(NCH_TOT, GPC, SUB)
  xp = jnp.pad(x, ((0, NP - N), (0, 0)))
  ea2 = jnp.concatenate([edge_attr, jnp.ones((E, DE), jnp.float32)], axis=1)
  z128 = jnp.zeros((NP, D), jnp.float32)

  eacnt = _sc_ea_cnt()(ea2, dst3d, z128)
  sx = _sc_spmm()(xp, src3d, dst3d, z128)
  h1 = _tc_node_stage(sx[0], sx[1], eacnt[0], eacnt[1], xp,
                      We1, be1, W1l, b1l, W1r)
  sh = _sc_spmm()(h1, src3d, dst3d, z128)
  h2, ps, pd = _tc_node_stage(sh[0], sh[1], eacnt[0], eacnt[1], h1,
                              We2, be2, W2l, b2l, W2r,
                              Wp_top=Wp1[:H], Wp_bot=Wp1[H:])
  q = _sc_edge_q()(ps, pd, src3d, dst3d)
  return _tc_pred_tail(q, bp1, Wp2, bp2)


# speedup vs baseline: 4.4170x; 1.1032x over previous
import functools

import jax
import jax.numpy as jnp
from jax import lax
from jax.experimental import pallas as pl
from jax.experimental.pallas import tpu as pltpu
from jax.experimental.pallas import tpu_sc as plsc

N = 10000
NP = 10240
E = 320000
D = 128
DE = 16
H = 128
C = 2

NC = 2
NS = 16
NW = NC * NS
EPT = E // NW
SUB = 40
GPC = 5
CHUNK = SUB * GPC
NCHUNK = EPT // CHUNK
NCH_TOT = E // CHUNK
RPT = NP // NS

_mesh = functools.partial(
    plsc.VectorSubcoreMesh, core_axis_name="c", subcore_axis_name="s")


def _sc_spmm():
  def body(table, src3d, dst3d, z128, sx_out, accx, sidx, didx, rows, sem):
    c = lax.axis_index("c")
    s = lax.axis_index("s")
    nbase = s * RPT
    pltpu.sync_copy(z128.at[pl.ds(nbase, RPT)], accx.at[pl.ds(nbase, RPT)])
    plsc.subcore_barrier()

    cbase = (c * NS + s) * NCHUNK

    @pl.loop(0, NCHUNK)
    def _(ch):
      cid = cbase + ch
      pltpu.sync_copy(src3d.at[cid], sidx)
      pltpu.sync_copy(dst3d.at[cid], didx)
      descs = [
          pltpu.async_copy(table.at[sidx.at[j]],
                           rows.at[pl.ds(j * SUB, SUB)], sem)
          for j in range(GPC)
      ]
      for d in descs:
        d.wait()
      for j in range(GPC):
        pltpu.sync_copy(rows.at[pl.ds(j * SUB, SUB)],
                        accx.at[didx.at[j]], add=True)

    plsc.subcore_barrier()
    pltpu.sync_copy(accx.at[pl.ds(nbase, RPT)],
                    sx_out.at[c, pl.ds(nbase, RPT)])

  return pl.kernel(
      body,
      out_type=jax.ShapeDtypeStruct((NC, NP, D), jnp.float32),
      mesh=_mesh(),
      scratch_types=[
          pltpu.VMEM_SHARED((NP, D), jnp.float32),
          pltpu.VMEM((GPC, SUB), jnp.int32),
          pltpu.VMEM((GPC, SUB), jnp.int32),
          pltpu.VMEM((CHUNK, D), jnp.float32),
          pltpu.SemaphoreType.DMA,
      ],
      name="sc_spmm")


EA2 = 2 * DE


def _sc_ea_cnt():
  def body(ea2, dst3d, z128, acc_out, acc, didx, eav, eaw, sem):
    c = lax.axis_index("c")
    s = lax.axis_index("s")
    nbase = s * RPT
    pltpu.sync_copy(z128.at[pl.ds(nbase, RPT)], acc.at[pl.ds(nbase, RPT)])
    pltpu.sync_copy(z128.at[pl.ds(0, SUB)], eaw)
    plsc.subcore_barrier()

    cbase = (c * NS + s) * NCHUNK

    @pl.loop(0, NCHUNK)
    def _(ch):
      cid = cbase + ch
      pltpu.sync_copy(dst3d.at[cid], didx)
      pltpu.sync_copy(ea2.at[pl.ds(cid * CHUNK, CHUNK)], eav)

      for j in range(GPC):
        @pl.loop(0, SUB // 4)
        def _(r4):
          for u in range(4):
            r = r4 * 4 + u
            e = j * SUB + r
            eaw[r, pl.ds(0, 16)] = eav[e, pl.ds(0, 16)]
            eaw[r, pl.ds(16, 16)] = eav[e, pl.ds(16, 16)]
        pltpu.sync_copy(eaw, acc.at[didx.at[j]], add=True)

    plsc.subcore_barrier()
    pltpu.sync_copy(acc.at[pl.ds(nbase, RPT)],
                    acc_out.at[c, pl.ds(nbase, RPT)])

  return pl.kernel(
      body,
      out_type=jax.ShapeDtypeStruct((NC, NP, D), jnp.float32),
      mesh=_mesh(),
      scratch_types=[
          pltpu.VMEM_SHARED((NP, D), jnp.float32),
          pltpu.VMEM((GPC, SUB), jnp.int32),
          pltpu.VMEM((CHUNK, EA2), jnp.float32),
          pltpu.VMEM((SUB, D), jnp.float32),
          pltpu.SemaphoreType.DMA,
      ],
      name="sc_ea_cnt")


def _sc_edge_q():
  def body(ps, pd, src1d, dst1d, q_out, sidx, didx, rows, sem0, sem1):
    c = lax.axis_index("c")
    s = lax.axis_index("s")
    wid = c * NS + s
    ebase = wid * EPT
    pltpu.sync_copy(src1d.at[pl.ds(ebase, EPT)], sidx)
    pltpu.sync_copy(dst1d.at[pl.ds(ebase, EPT)], didx)
    sems = (sem0, sem1)

    def fire_g(ch, b):
      for j in range(GPC):
        pltpu.async_copy(
            ps.at[sidx.at[pl.ds(ch * CHUNK + j * SUB, SUB)]],
            rows.at[b].at[pl.ds(j * SUB, SUB)], sems[b])

    def fire_a(ch, b):
      for j in range(GPC):
        pltpu.async_copy(
            pd.at[didx.at[pl.ds(ch * CHUNK + j * SUB, SUB)]],
            rows.at[b].at[pl.ds(j * SUB, SUB)], sems[b], add=True)

    def drain(b, nbytes_rows):
      pltpu.make_async_copy(
          q_out.at[pl.ds(0, nbytes_rows)], rows.at[b].at[pl.ds(0, nbytes_rows)],
          sems[b]).wait()

    def drain_w(ch, b):
      pltpu.make_async_copy(
          rows.at[b], q_out.at[pl.ds(ch * CHUNK, CHUNK)], sems[b]).wait()

    fire_g(0, 0)

    def half(ch, b):
      drain(b, CHUNK)
      fire_a(ch, b)

      @pl.when(ch + 1 < NCHUNK)
      def _():
        @pl.when(ch >= 1)
        def _():
          drain_w(ch - 1, 1 - b)
        fire_g(ch + 1, 1 - b)

      drain(b, CHUNK)
      pltpu.async_copy(rows.at[b],
                       q_out.at[pl.ds((ebase + ch * CHUNK), CHUNK)], sems[b])

    @pl.loop(0, NCHUNK // 2)
    def _(k):
      half(2 * k, 0)
      half(2 * k + 1, 1)

    drain_w(NCHUNK - 2, 0)
    drain_w(NCHUNK - 1, 1)

  return pl.kernel(
      body,
      out_type=jax.ShapeDtypeStruct((E, H), jnp.float32),
      mesh=_mesh(),
      scratch_types=[
          pltpu.VMEM((EPT,), jnp.int32),
          pltpu.VMEM((EPT,), jnp.int32),
          pltpu.VMEM((2, CHUNK, H), jnp.float32),
          pltpu.SemaphoreType.DMA,
          pltpu.SemaphoreType.DMA,
      ],
      name="sc_edge_q")


def _tc_node_stage(sx0, sx1, eacnt0, eacnt1, x_in, We, be, Wl, bl, Wr,
                   Wp_top=None, Wp_bot=None):
  with_proj = Wp_top is not None

  def body(*refs):
    if with_proj:
      (sx0r, sx1r, e0r, e1r, xr, Wer, ber, Wlr, blr, Wrr,
       Wtr, Wbr, hr, psr, pdr) = refs
    else:
      (sx0r, sx1r, e0r, e1r, xr, Wer, ber, Wlr, blr, Wrr, hr) = refs
    eacnt = e0r[...] + e1r[...]
    Ea = eacnt[:, :DE]
    cnt = eacnt[:, DE:DE + 1]
    denom = jnp.maximum(cnt, 1.0)
    S = sx0r[...] + sx1r[...]
    msum = S + jnp.dot(Ea, Wer[...],
                       preferred_element_type=jnp.float32) + cnt * ber[...]
    agg = msum / denom
    h = jnp.maximum(
        jnp.dot(agg, Wlr[...], preferred_element_type=jnp.float32)
        + blr[...]
        + jnp.dot(xr[...], Wrr[...], preferred_element_type=jnp.float32),
        0.0)
    hr[...] = h
    if with_proj:
      psr[...] = jnp.dot(h, Wtr[...], preferred_element_type=jnp.float32)
      pdr[...] = jnp.dot(h, Wbr[...], preferred_element_type=jnp.float32)

  out_shape = jax.ShapeDtypeStruct((NP, H), jnp.float32)
  if with_proj:
    out_shape = (out_shape,
                 jax.ShapeDtypeStruct((NP, H), jnp.float32),
                 jax.ShapeDtypeStruct((NP, H), jnp.float32))
  args = [sx0, sx1, eacnt0, eacnt1, x_in, We, be, Wl, bl, Wr]
  if with_proj:
    args += [Wp_top, Wp_bot]
  return pl.pallas_call(body, out_shape=out_shape,
                        name="tc_node_proj" if with_proj else "tc_node")(*args)


_EB = 8000


def _tc_pred_tail(q, bp1, Wp2, bp2):
  def body(qr, b1r, W2r, b2r, outr):
    z = jnp.maximum(qr[...] + b1r[...], 0.0)
    outr[...] = jnp.dot(z, W2r[...],
                        preferred_element_type=jnp.float32) + b2r[...]

  return pl.pallas_call(
      body,
      grid=(E // _EB,),
      in_specs=[
          pl.BlockSpec((_EB, H), lambda i: (i, 0)),
          pl.BlockSpec((H,), lambda i: (0,)),
          pl.BlockSpec((H, C), lambda i: (0, 0)),
          pl.BlockSpec((C,), lambda i: (0,)),
      ],
      out_specs=pl.BlockSpec((_EB, C), lambda i: (i, 0)),
      out_shape=jax.ShapeDtypeStruct((E, C), jnp.float32),
      name="tc_pred_tail")(q, bp1, Wp2, bp2)


@jax.jit
def kernel(x, edge_index, edge_attr, We1, be1, W1l, b1l, W1r,
           We2, be2, W2l, b2l, W2r, Wp1, bp1, Wp2, bp2):
  src1d = edge_index[0].astype(jnp.int32)
  dst1d = edge_index[1].astype(jnp.int32)
  src3d = src1d.reshape(NCH_TOT, GPC, SUB)
  dst3d = dst1d.reshape(NCH_TOT, GPC, SUB)
  xp = jnp.pad(x, ((0, NP - N), (0, 0)))
  ea2 = jnp.concatenate([edge_attr, jnp.ones((E, DE), jnp.float32)], axis=1)
  z128 = jnp.zeros((NP, D), jnp.float32)

  eacnt = _sc_ea_cnt()(ea2, dst3d, z128)
  sx = _sc_spmm()(xp, src3d, dst3d, z128)
  h1 = _tc_node_stage(sx[0], sx[1], eacnt[0], eacnt[1], xp,
                      We1, be1, W1l, b1l, W1r)
  sh = _sc_spmm()(h1, src3d, dst3d, z128)
  h2, ps, pd = _tc_node_stage(sh[0], sh[1], eacnt[0], eacnt[1], h1,
                              We2, be2, W2l, b2l, W2r,
                              Wp_top=Wp1[:H], Wp_bot=Wp1[H:])
  q = _sc_edge_q()(ps, pd, src1d, dst1d)
  return _tc_pred_tail(q, bp1, Wp2, bp2)


# speedup vs baseline: 4.8733x; 1.1033x over previous
import functools

import jax
import jax.numpy as jnp
from jax import lax
from jax.experimental import pallas as pl
from jax.experimental.pallas import tpu as pltpu
from jax.experimental.pallas import tpu_sc as plsc

N = 10000
NP = 10240
E = 320000
D = 128
DE = 16
H = 128
C = 2

NC = 2
NS = 16
NW = NC * NS
EPT = E // NW
SUB = 40
GPC = 5
CHUNK = SUB * GPC
NCHUNK = EPT // CHUNK
NCH_TOT = E // CHUNK
RPT = NP // NS

_mesh = functools.partial(
    plsc.VectorSubcoreMesh, core_axis_name="c", subcore_axis_name="s")


SCH = 80
NSCH = EPT // SCH


def _sc_spmm():
  def body(table, src1d, dst1d, z128, sx_out,
           accx, sidx, didx, rows, semg0, semg1, sems0, sems1):
    c = lax.axis_index("c")
    s = lax.axis_index("s")
    nbase = s * RPT
    pltpu.sync_copy(z128.at[pl.ds(nbase, RPT)], accx.at[pl.ds(nbase, RPT)])
    ebase = (c * NS + s) * EPT
    pltpu.sync_copy(src1d.at[pl.ds(ebase, EPT)], sidx)
    pltpu.sync_copy(dst1d.at[pl.ds(ebase, EPT)], didx)
    plsc.subcore_barrier()

    semg = (semg0, semg1)
    sems = (sems0, sems1)

    def fire_g(ch, b):
      pltpu.async_copy(table.at[sidx.at[pl.ds(ch * SCH, SCH)]],
                       rows.at[b], semg[b])

    def drain_g(ch, b):
      pltpu.make_async_copy(z128.at[pl.ds(0, SCH)], rows.at[b],
                            semg[b]).wait()

    def fire_s(ch, b):
      for k in range(SCH // 16):
        dv = didx[pl.ds(ch * SCH + k * 16, 16)]
        pltpu.async_copy(rows.at[b].at[pl.ds(k * 16, 16)],
                         accx.at[dv], sems[b], add=True)

    def drain_s(b):
      pltpu.make_async_copy(z128.at[pl.ds(0, SCH)], rows.at[b],
                            sems[b]).wait()

    fire_g(0, 0)

    def half(ch, b):
      drain_g(ch, b)
      fire_s(ch, b)

      @pl.when(ch + 1 < NSCH)
      def _():
        @pl.when(ch >= 1)
        def _():
          drain_s(1 - b)
        fire_g(ch + 1, 1 - b)

    @pl.loop(0, NSCH // 2)
    def _(k):
      half(2 * k, 0)
      half(2 * k + 1, 1)

    half(NSCH - 1, 0)
    drain_s(1)
    drain_s(0)

    plsc.subcore_barrier()
    pltpu.sync_copy(accx.at[pl.ds(nbase, RPT)],
                    sx_out.at[c, pl.ds(nbase, RPT)])

  return pl.kernel(
      body,
      out_type=jax.ShapeDtypeStruct((NC, NP, D), jnp.float32),
      mesh=_mesh(),
      scratch_types=[
          pltpu.VMEM_SHARED((NP, D), jnp.float32),
          pltpu.VMEM((EPT,), jnp.int32),
          pltpu.VMEM((EPT,), jnp.int32),
          pltpu.VMEM((2, SCH, D), jnp.float32),
          pltpu.SemaphoreType.DMA,
          pltpu.SemaphoreType.DMA,
          pltpu.SemaphoreType.DMA,
          pltpu.SemaphoreType.DMA,
      ],
      name="sc_spmm")


EA2 = 2 * DE


def _sc_ea_cnt():
  def body(ea2, dst3d, z128, acc_out, acc, didx, eav, eaw, sem):
    c = lax.axis_index("c")
    s = lax.axis_index("s")
    nbase = s * RPT
    pltpu.sync_copy(z128.at[pl.ds(nbase, RPT)], acc.at[pl.ds(nbase, RPT)])
    pltpu.sync_copy(z128.at[pl.ds(0, SUB)], eaw)
    plsc.subcore_barrier()

    cbase = (c * NS + s) * NCHUNK

    @pl.loop(0, NCHUNK)
    def _(ch):
      cid = cbase + ch
      pltpu.sync_copy(dst3d.at[cid], didx)
      pltpu.sync_copy(ea2.at[pl.ds(cid * CHUNK, CHUNK)], eav)

      for j in range(GPC):
        @pl.loop(0, SUB // 4)
        def _(r4):
          for u in range(4):
            r = r4 * 4 + u
            e = j * SUB + r
            eaw[r, pl.ds(0, 16)] = eav[e, pl.ds(0, 16)]
            eaw[r, pl.ds(16, 16)] = eav[e, pl.ds(16, 16)]
        pltpu.sync_copy(eaw, acc.at[didx.at[j]], add=True)

    plsc.subcore_barrier()
    pltpu.sync_copy(acc.at[pl.ds(nbase, RPT)],
                    acc_out.at[c, pl.ds(nbase, RPT)])

  return pl.kernel(
      body,
      out_type=jax.ShapeDtypeStruct((NC, NP, D), jnp.float32),
      mesh=_mesh(),
      scratch_types=[
          pltpu.VMEM_SHARED((NP, D), jnp.float32),
          pltpu.VMEM((GPC, SUB), jnp.int32),
          pltpu.VMEM((CHUNK, EA2), jnp.float32),
          pltpu.VMEM((SUB, D), jnp.float32),
          pltpu.SemaphoreType.DMA,
      ],
      name="sc_ea_cnt")


def _sc_edge_q():
  def body(ps, pd, src1d, dst1d, q_out, sidx, didx, rows, sem0, sem1):
    c = lax.axis_index("c")
    s = lax.axis_index("s")
    wid = c * NS + s
    ebase = wid * EPT
    pltpu.sync_copy(src1d.at[pl.ds(ebase, EPT)], sidx)
    pltpu.sync_copy(dst1d.at[pl.ds(ebase, EPT)], didx)
    sems = (sem0, sem1)

    def fire_g(ch, b):
      for j in range(GPC):
        pltpu.async_copy(
            ps.at[sidx.at[pl.ds(ch * CHUNK + j * SUB, SUB)]],
            rows.at[b].at[pl.ds(j * SUB, SUB)], sems[b])

    def fire_a(ch, b):
      for j in range(GPC):
        pltpu.async_copy(
            pd.at[didx.at[pl.ds(ch * CHUNK + j * SUB, SUB)]],
            rows.at[b].at[pl.ds(j * SUB, SUB)], sems[b], add=True)

    def drain(b, nbytes_rows):
      pltpu.make_async_copy(
          q_out.at[pl.ds(0, nbytes_rows)], rows.at[b].at[pl.ds(0, nbytes_rows)],
          sems[b]).wait()

    def drain_w(ch, b):
      pltpu.make_async_copy(
          rows.at[b], q_out.at[pl.ds(ch * CHUNK, CHUNK)], sems[b]).wait()

    fire_g(0, 0)

    def half(ch, b):
      drain(b, CHUNK)
      fire_a(ch, b)

      @pl.when(ch + 1 < NCHUNK)
      def _():
        @pl.when(ch >= 1)
        def _():
          drain_w(ch - 1, 1 - b)
        fire_g(ch + 1, 1 - b)

      drain(b, CHUNK)
      pltpu.async_copy(rows.at[b],
                       q_out.at[pl.ds((ebase + ch * CHUNK), CHUNK)], sems[b])

    @pl.loop(0, NCHUNK // 2)
    def _(k):
      half(2 * k, 0)
      half(2 * k + 1, 1)

    drain_w(NCHUNK - 2, 0)
    drain_w(NCHUNK - 1, 1)

  return pl.kernel(
      body,
      out_type=jax.ShapeDtypeStruct((E, H), jnp.float32),
      mesh=_mesh(),
      scratch_types=[
          pltpu.VMEM((EPT,), jnp.int32),
          pltpu.VMEM((EPT,), jnp.int32),
          pltpu.VMEM((2, CHUNK, H), jnp.float32),
          pltpu.SemaphoreType.DMA,
          pltpu.SemaphoreType.DMA,
      ],
      name="sc_edge_q")


def _tc_node_stage(sx0, sx1, eacnt0, eacnt1, x_in, We, be, Wl, bl, Wr,
                   Wp_top=None, Wp_bot=None):
  with_proj = Wp_top is not None

  def body(*refs):
    if with_proj:
      (sx0r, sx1r, e0r, e1r, xr, Wer, ber, Wlr, blr, Wrr,
       Wtr, Wbr, hr, psr, pdr) = refs
    else:
      (sx0r, sx1r, e0r, e1r, xr, Wer, ber, Wlr, blr, Wrr, hr) = refs
    eacnt = e0r[...] + e1r[...]
    Ea = eacnt[:, :DE]
    cnt = eacnt[:, DE:DE + 1]
    denom = jnp.maximum(cnt, 1.0)
    S = sx0r[...] + sx1r[...]
    msum = S + jnp.dot(Ea, Wer[...],
                       preferred_element_type=jnp.float32) + cnt * ber[...]
    agg = msum / denom
    h = jnp.maximum(
        jnp.dot(agg, Wlr[...], preferred_element_type=jnp.float32)
        + blr[...]
        + jnp.dot(xr[...], Wrr[...], preferred_element_type=jnp.float32),
        0.0)
    hr[...] = h
    if with_proj:
      psr[...] = jnp.dot(h, Wtr[...], preferred_element_type=jnp.float32)
      pdr[...] = jnp.dot(h, Wbr[...], preferred_element_type=jnp.float32)

  out_shape = jax.ShapeDtypeStruct((NP, H), jnp.float32)
  if with_proj:
    out_shape = (out_shape,
                 jax.ShapeDtypeStruct((NP, H), jnp.float32),
                 jax.ShapeDtypeStruct((NP, H), jnp.float32))
  args = [sx0, sx1, eacnt0, eacnt1, x_in, We, be, Wl, bl, Wr]
  if with_proj:
    args += [Wp_top, Wp_bot]
  return pl.pallas_call(body, out_shape=out_shape,
                        name="tc_node_proj" if with_proj else "tc_node")(*args)


_EB = 8000


def _tc_pred_tail(q, bp1, Wp2, bp2):
  def body(qr, b1r, W2r, b2r, outr):
    z = jnp.maximum(qr[...] + b1r[...], 0.0)
    outr[...] = jnp.dot(z, W2r[...],
                        preferred_element_type=jnp.float32) + b2r[...]

  return pl.pallas_call(
      body,
      grid=(E // _EB,),
      in_specs=[
          pl.BlockSpec((_EB, H), lambda i: (i, 0)),
          pl.BlockSpec((H,), lambda i: (0,)),
          pl.BlockSpec((H, C), lambda i: (0, 0)),
          pl.BlockSpec((C,), lambda i: (0,)),
      ],
      out_specs=pl.BlockSpec((_EB, C), lambda i: (i, 0)),
      out_shape=jax.ShapeDtypeStruct((E, C), jnp.float32),
      name="tc_pred_tail")(q, bp1, Wp2, bp2)


@jax.jit
def kernel(x, edge_index, edge_attr, We1, be1, W1l, b1l, W1r,
           We2, be2, W2l, b2l, W2r, Wp1, bp1, Wp2, bp2):
  src1d = edge_index[0].astype(jnp.int32)
  dst1d = edge_index[1].astype(jnp.int32)
  src3d = src1d.reshape(NCH_TOT, GPC, SUB)
  dst3d = dst1d.reshape(NCH_TOT, GPC, SUB)
  xp = jnp.pad(x, ((0, NP - N), (0, 0)))
  ea2 = jnp.concatenate([edge_attr, jnp.ones((E, DE), jnp.float32)], axis=1)
  z128 = jnp.zeros((NP, D), jnp.float32)

  eacnt = _sc_ea_cnt()(ea2, dst3d, z128)
  sx = _sc_spmm()(xp, src1d, dst1d, z128)
  h1 = _tc_node_stage(sx[0], sx[1], eacnt[0], eacnt[1], xp,
                      We1, be1, W1l, b1l, W1r)
  sh = _sc_spmm()(h1, src1d, dst1d, z128)
  h2, ps, pd = _tc_node_stage(sh[0], sh[1], eacnt[0], eacnt[1], h1,
                              We2, be2, W2l, b2l, W2r,
                              Wp_top=Wp1[:H], Wp_bot=Wp1[H:])
  q = _sc_edge_q()(ps, pd, src1d, dst1d)
  return _tc_pred_tail(q, bp1, Wp2, bp2)
